# Initial kernel scaffold; baseline (speedup 1.0000x reference)
import functools

import jax
import jax.numpy as jnp
from jax import lax
from jax.experimental import pallas as pl
from jax.experimental.pallas import tpu as pltpu
from jax.experimental.pallas import tpu_sc as plsc

NC = 2
NS = 16
CH = 80


def _sc_scatter_add(e, ridx, zeros_nh, n_nodes):
    n_edges, h = e.shape
    per_core = n_edges // NC
    per_sub = per_core // NS
    n_chunks = per_sub // CH
    rows_per_sub = n_nodes // NS

    mesh = plsc.VectorSubcoreMesh(
        core_axis_name="c", subcore_axis_name="s", num_cores=NC, num_subcores=NS
    )

    @functools.partial(
        pl.kernel,
        out_type=jax.ShapeDtypeStruct((NC, n_nodes, h), jnp.float32),
        mesh=mesh,
        scratch_types=[
            pltpu.VMEM_SHARED((n_nodes, h), jnp.float32),
            pltpu.VMEM((CH, h), jnp.float32),
            pltpu.VMEM((CH,), jnp.int32),
        ],
    )
    def sc_kernel(e_hbm, ridx_hbm, zeros_hbm, out_hbm, acc_sh, ebuf, idxbuf):
        c = lax.axis_index("c")
        s = lax.axis_index("s")
        base_e = c * per_core + s * per_sub
        row0 = s * rows_per_sub

        pltpu.sync_copy(
            zeros_hbm.at[pl.ds(row0, rows_per_sub)],
            acc_sh.at[pl.ds(row0, rows_per_sub)],
        )
        plsc.subcore_barrier()

        def body(j, _):
            off = base_e + j * CH
            pltpu.sync_copy(ridx_hbm.at[pl.ds(off, CH)], idxbuf)
            pltpu.sync_copy(e_hbm.at[pl.ds(off, CH)], ebuf)
            pltpu.sync_copy(ebuf, acc_sh.at[idxbuf], add=True)
            return 0

        lax.fori_loop(0, n_chunks, body, 0)
        plsc.subcore_barrier()

        pltpu.sync_copy(
            acc_sh.at[pl.ds(row0, rows_per_sub)],
            out_hbm.at[c, pl.ds(row0, rows_per_sub)],
        )

    return sc_kernel(e, ridx, zeros_nh)


def _tc_mlp(partials, v, W_e, W_v, b0, W1, b1, gamma, beta, block_rows=500):
    n, h = v.shape
    grid = (n // block_rows,)

    def body(p_ref, v_ref, we_ref, wv_ref, b0_ref, w1_ref, b1_ref, g_ref,
             bt_ref, o_ref):
        agg = p_ref[0] + p_ref[1]
        x = (
            jnp.dot(agg, we_ref[...], preferred_element_type=jnp.float32)
            + jnp.dot(v_ref[...], wv_ref[...], preferred_element_type=jnp.float32)
            + b0_ref[...]
        )
        x = jnp.maximum(x, 0.0)
        x = jnp.dot(x, w1_ref[...], preferred_element_type=jnp.float32) + b1_ref[...]
        x = jnp.maximum(x, 0.0)
        mu = jnp.mean(x, axis=-1, keepdims=True)
        xc = x - mu
        var = jnp.mean(xc * xc, axis=-1, keepdims=True)
        o_ref[...] = xc * jax.lax.rsqrt(var + 1e-5) * g_ref[...] + bt_ref[...]

    full = lambda i: (0, 0)
    return pl.pallas_call(
        body,
        grid=grid,
        in_specs=[
            pl.BlockSpec((NC, block_rows, h), lambda i: (0, i, 0)),
            pl.BlockSpec((block_rows, h), lambda i: (i, 0)),
            pl.BlockSpec((h, h), full),
            pl.BlockSpec((h, h), full),
            pl.BlockSpec((1, h), full),
            pl.BlockSpec((h, h), full),
            pl.BlockSpec((1, h), full),
            pl.BlockSpec((1, h), full),
            pl.BlockSpec((1, h), full),
        ],
        out_specs=pl.BlockSpec((block_rows, h), lambda i: (i, 0)),
        out_shape=jax.ShapeDtypeStruct((n, h), jnp.float32),
    )(partials, v, W_e, W_v, b0.reshape(1, h), W1, b1.reshape(1, h),
      gamma.reshape(1, h), beta.reshape(1, h))


@jax.jit
def kernel(v, edge_index, e, W_e, W_v, b0, W1, b1, gamma, beta):
    n, h = v.shape
    ridx = edge_index[1].astype(jnp.int32)
    zeros_nh = jnp.zeros((n, h), jnp.float32)
    partials = _sc_scatter_add(e, ridx, zeros_nh, n)
    return _tc_mlp(partials, v, W_e, W_v, b0, W1, b1, gamma, beta)


# speedup vs baseline: 3.5140x; 3.5140x over previous
import functools

import jax
import jax.numpy as jnp
from jax import lax
from jax.experimental import pallas as pl
from jax.experimental.pallas import tpu as pltpu
from jax.experimental.pallas import tpu_sc as plsc

NC = 2
NS = 16
CH = 80


def _sc_scatter_add(e, ridx, zeros_nh, n_nodes):
    n_edges, h = e.shape
    per_core = n_edges // NC
    per_sub = per_core // NS
    n_chunks = per_sub // CH
    n_pad = ((n_nodes + NS * 8 - 1) // (NS * 8)) * (NS * 8)
    rows_per_sub = n_pad // NS

    mesh = plsc.VectorSubcoreMesh(
        core_axis_name="c", subcore_axis_name="s", num_cores=NC, num_subcores=NS
    )

    @functools.partial(
        pl.kernel,
        out_type=jax.ShapeDtypeStruct((NC, n_pad, h), jnp.float32),
        mesh=mesh,
        scratch_types=[
            pltpu.VMEM_SHARED((n_pad, h), jnp.float32),
            pltpu.VMEM((CH, h), jnp.float32),
            pltpu.VMEM((CH,), jnp.int32),
        ],
    )
    def sc_kernel(e_hbm, ridx_hbm, zeros_hbm, out_hbm, acc_sh, ebuf, idxbuf):
        c = lax.axis_index("c")
        s = lax.axis_index("s")
        base_e = c * per_core + s * per_sub
        row0 = s * rows_per_sub

        pltpu.sync_copy(
            zeros_hbm.at[pl.ds(row0, rows_per_sub)],
            acc_sh.at[pl.ds(row0, rows_per_sub)],
        )
        plsc.subcore_barrier()

        def body(j, _):
            off = base_e + j * CH
            pltpu.sync_copy(ridx_hbm.at[pl.ds(off, CH)], idxbuf)
            pltpu.sync_copy(e_hbm.at[pl.ds(off, CH)], ebuf)
            pltpu.sync_copy(ebuf, acc_sh.at[idxbuf], add=True)
            return 0

        lax.fori_loop(0, n_chunks, body, 0)
        plsc.subcore_barrier()

        pltpu.sync_copy(
            acc_sh.at[pl.ds(row0, rows_per_sub)],
            out_hbm.at[c, pl.ds(row0, rows_per_sub)],
        )

    return sc_kernel(e, ridx, zeros_nh)[:, :n_nodes, :]


def _tc_mlp(partials, v, W_e, W_v, b0, W1, b1, gamma, beta, block_rows=400):
    n, h = v.shape
    grid = (n // block_rows,)

    def body(p_ref, v_ref, we_ref, wv_ref, b0_ref, w1_ref, b1_ref, g_ref,
             bt_ref, o_ref):
        agg = p_ref[0] + p_ref[1]
        x = (
            jnp.dot(agg, we_ref[...], preferred_element_type=jnp.float32)
            + jnp.dot(v_ref[...], wv_ref[...], preferred_element_type=jnp.float32)
            + b0_ref[...]
        )
        x = jnp.maximum(x, 0.0)
        x = jnp.dot(x, w1_ref[...], preferred_element_type=jnp.float32) + b1_ref[...]
        x = jnp.maximum(x, 0.0)
        mu = jnp.mean(x, axis=-1, keepdims=True)
        xc = x - mu
        var = jnp.mean(xc * xc, axis=-1, keepdims=True)
        o_ref[...] = xc * jax.lax.rsqrt(var + 1e-5) * g_ref[...] + bt_ref[...]

    full = lambda i: (0, 0)
    return pl.pallas_call(
        body,
        grid=grid,
        in_specs=[
            pl.BlockSpec((NC, block_rows, h), lambda i: (0, i, 0)),
            pl.BlockSpec((block_rows, h), lambda i: (i, 0)),
            pl.BlockSpec((h, h), full),
            pl.BlockSpec((h, h), full),
            pl.BlockSpec((1, h), full),
            pl.BlockSpec((h, h), full),
            pl.BlockSpec((1, h), full),
            pl.BlockSpec((1, h), full),
            pl.BlockSpec((1, h), full),
        ],
        out_specs=pl.BlockSpec((block_rows, h), lambda i: (i, 0)),
        out_shape=jax.ShapeDtypeStruct((n, h), jnp.float32),
    )(partials, v, W_e, W_v, b0.reshape(1, h), W1, b1.reshape(1, h),
      gamma.reshape(1, h), beta.reshape(1, h))


@jax.jit
def kernel(v, edge_index, e, W_e, W_v, b0, W1, b1, gamma, beta):
    n, h = v.shape
    ridx = edge_index[1].astype(jnp.int32)
    n_pad = ((n + NS * 8 - 1) // (NS * 8)) * (NS * 8)
    zeros_nh = jnp.zeros((n_pad, h), jnp.float32)
    partials = _sc_scatter_add(e, ridx, zeros_nh, n)
    return _tc_mlp(partials, v, W_e, W_v, b0, W1, b1, gamma, beta)


# speedup vs baseline: 6.8350x; 1.9450x over previous
import functools

import jax
import jax.numpy as jnp
from jax import lax
from jax.experimental import pallas as pl
from jax.experimental.pallas import tpu as pltpu
from jax.experimental.pallas import tpu_sc as plsc

NC = 2
NS = 16
CH = 80


NBUF = 3


def _sc_scatter_add(e, ridx4, zeros_nh, n_nodes):
    n_edges, h = e.shape
    per_sub = n_edges // (NC * NS)
    n_chunks = per_sub // CH
    n_pad = ((n_nodes + NS * 8 - 1) // (NS * 8)) * (NS * 8)
    rows_per_sub = n_pad // NS

    mesh = plsc.VectorSubcoreMesh(
        core_axis_name="c", subcore_axis_name="s", num_cores=NC, num_subcores=NS
    )

    @functools.partial(
        pl.kernel,
        out_type=jax.ShapeDtypeStruct((NC, n_pad, h), jnp.float32),
        mesh=mesh,
        scratch_types=[
            pltpu.VMEM_SHARED((n_pad, h), jnp.float32),
            pltpu.VMEM((NBUF, CH, h), jnp.float32),
            pltpu.VMEM((n_chunks, CH), jnp.int32),
            pltpu.SemaphoreType.DMA,
            pltpu.SemaphoreType.DMA,
        ],
    )
    def sc_kernel(e_hbm, ridx_hbm, zeros_hbm, out_hbm, acc_sh, bbuf, idx_all,
                  lsem, ssem):
        c = lax.axis_index("c")
        s = lax.axis_index("s")
        base_e = (c * NS + s) * per_sub
        row0 = s * rows_per_sub

        def start_load(k, buf):
            pltpu.async_copy(
                e_hbm.at[pl.ds(base_e + k * CH, CH)], bbuf.at[buf], lsem
            )

        def wait_load(buf):
            pltpu.make_async_copy(
                e_hbm.at[pl.ds(base_e, CH)], bbuf.at[buf], lsem
            ).wait()

        def wait_scatter_one():
            pltpu.make_async_copy(
                e_hbm.at[pl.ds(base_e, CH)], bbuf.at[0], ssem
            ).wait()

        start_load(0, 0)
        start_load(1, 1)
        pltpu.sync_copy(
            zeros_hbm.at[pl.ds(row0, rows_per_sub)],
            acc_sh.at[pl.ds(row0, rows_per_sub)],
        )
        pltpu.sync_copy(ridx_hbm.at[c, s], idx_all)
        plsc.subcore_barrier()

        def body(k, _):
            buf = lax.rem(k, NBUF)
            wait_load(buf)

            @pl.when(k >= 2)
            def _():
                wait_scatter_one()

            @pl.when(k + 2 < n_chunks)
            def _():
                start_load(k + 2, lax.rem(k + 2, NBUF))

            pltpu.async_copy(
                bbuf.at[buf], acc_sh.at[idx_all.at[k]], ssem, add=True
            )
            return 0

        lax.fori_loop(0, n_chunks, body, 0)
        wait_scatter_one()
        wait_scatter_one()
        plsc.subcore_barrier()

        pltpu.sync_copy(
            acc_sh.at[pl.ds(row0, rows_per_sub)],
            out_hbm.at[c, pl.ds(row0, rows_per_sub)],
        )

    return sc_kernel(e, ridx4, zeros_nh)[:, :n_nodes, :]


def _tc_mlp(partials, v, W_e, W_v, b0, W1, b1, gamma, beta, block_rows=400):
    n, h = v.shape
    grid = (n // block_rows,)

    def body(p_ref, v_ref, we_ref, wv_ref, b0_ref, w1_ref, b1_ref, g_ref,
             bt_ref, o_ref):
        agg = p_ref[0] + p_ref[1]
        x = (
            jnp.dot(agg, we_ref[...], preferred_element_type=jnp.float32)
            + jnp.dot(v_ref[...], wv_ref[...], preferred_element_type=jnp.float32)
            + b0_ref[...]
        )
        x = jnp.maximum(x, 0.0)
        x = jnp.dot(x, w1_ref[...], preferred_element_type=jnp.float32) + b1_ref[...]
        x = jnp.maximum(x, 0.0)
        mu = jnp.mean(x, axis=-1, keepdims=True)
        xc = x - mu
        var = jnp.mean(xc * xc, axis=-1, keepdims=True)
        o_ref[...] = xc * jax.lax.rsqrt(var + 1e-5) * g_ref[...] + bt_ref[...]

    full = lambda i: (0, 0)
    return pl.pallas_call(
        body,
        grid=grid,
        in_specs=[
            pl.BlockSpec((NC, block_rows, h), lambda i: (0, i, 0)),
            pl.BlockSpec((block_rows, h), lambda i: (i, 0)),
            pl.BlockSpec((h, h), full),
            pl.BlockSpec((h, h), full),
            pl.BlockSpec((1, h), full),
            pl.BlockSpec((h, h), full),
            pl.BlockSpec((1, h), full),
            pl.BlockSpec((1, h), full),
            pl.BlockSpec((1, h), full),
        ],
        out_specs=pl.BlockSpec((block_rows, h), lambda i: (i, 0)),
        out_shape=jax.ShapeDtypeStruct((n, h), jnp.float32),
    )(partials, v, W_e, W_v, b0.reshape(1, h), W1, b1.reshape(1, h),
      gamma.reshape(1, h), beta.reshape(1, h))


@jax.jit
def kernel(v, edge_index, e, W_e, W_v, b0, W1, b1, gamma, beta):
    n, h = v.shape
    n_edges = e.shape[0]
    per_sub = n_edges // (NC * NS)
    ridx4 = edge_index[1].astype(jnp.int32).reshape(NC, NS, per_sub // CH, CH)
    n_pad = ((n + NS * 8 - 1) // (NS * 8)) * (NS * 8)
    zeros_nh = jnp.zeros((n_pad, h), jnp.float32)
    partials = _sc_scatter_add(e, ridx4, zeros_nh, n)
    return _tc_mlp(partials, v, W_e, W_v, b0, W1, b1, gamma, beta)


# speedup vs baseline: 7.6041x; 1.1125x over previous
import functools

import jax
import jax.numpy as jnp
import numpy as np
from jax import lax
from jax.experimental import pallas as pl
from jax.experimental.pallas import tpu as pltpu
from jax.experimental.pallas import tpu_sc as plsc

NC = 2
NS = 16
CH = 80


NBUF = 3


def _sc_scatter_add(e, ridx4, zeros_nh, n_nodes):
    n_edges, h = e.shape
    per_sub = n_edges // (NC * NS)
    n_chunks = per_sub // CH
    n_pad = ((n_nodes + NS * 8 - 1) // (NS * 8)) * (NS * 8)
    rows_per_sub = n_pad // NS

    mesh = plsc.VectorSubcoreMesh(
        core_axis_name="c", subcore_axis_name="s", num_cores=NC, num_subcores=NS
    )

    @functools.partial(
        pl.kernel,
        out_type=jax.ShapeDtypeStruct((NC, n_pad, h), jnp.float32),
        mesh=mesh,
        scratch_types=[
            pltpu.VMEM_SHARED((n_pad, h), jnp.float32),
            pltpu.VMEM((NBUF, CH, h), jnp.float32),
            pltpu.VMEM((n_chunks, CH), jnp.int32),
            pltpu.SemaphoreType.DMA,
            pltpu.SemaphoreType.DMA,
        ],
    )
    def sc_kernel(e_hbm, ridx_hbm, zeros_hbm, out_hbm, acc_sh, bbuf, idx_all,
                  lsem, ssem):
        c = lax.axis_index("c")
        s = lax.axis_index("s")
        base_e = (c * NS + s) * per_sub
        row0 = s * rows_per_sub

        def start_load(k, buf):
            pltpu.async_copy(
                e_hbm.at[pl.ds(base_e + k * CH, CH)], bbuf.at[buf], lsem
            )

        def wait_load(buf):
            pltpu.make_async_copy(
                e_hbm.at[pl.ds(base_e, CH)], bbuf.at[buf], lsem
            ).wait()

        def wait_scatter_one():
            pltpu.make_async_copy(
                e_hbm.at[pl.ds(base_e, CH)], bbuf.at[0], ssem
            ).wait()

        start_load(0, 0)
        start_load(1, 1)
        pltpu.sync_copy(
            zeros_hbm.at[pl.ds(row0, rows_per_sub)],
            acc_sh.at[pl.ds(row0, rows_per_sub)],
        )
        pltpu.sync_copy(ridx_hbm.at[c, s], idx_all)
        plsc.subcore_barrier()

        def body(k, _):
            buf = lax.rem(k, NBUF)
            wait_load(buf)

            @pl.when(k >= 2)
            def _():
                wait_scatter_one()

            @pl.when(k + 2 < n_chunks)
            def _():
                start_load(k + 2, lax.rem(k + 2, NBUF))

            pltpu.async_copy(
                bbuf.at[buf], acc_sh.at[idx_all.at[k]], ssem, add=True
            )
            return 0

        lax.fori_loop(0, n_chunks, body, 0)
        wait_scatter_one()
        wait_scatter_one()
        plsc.subcore_barrier()

        pltpu.sync_copy(
            acc_sh.at[pl.ds(row0, rows_per_sub)],
            out_hbm.at[c, pl.ds(row0, rows_per_sub)],
        )

    return sc_kernel(e, ridx4, zeros_nh)


def _tc_mlp(partials, v, W_e, W_v, b0, W1, b1, gamma, beta, block_rows=1000):
    n, h = v.shape
    grid = (n // block_rows,)

    def body(p_ref, v_ref, we_ref, wv_ref, b0_ref, w1_ref, b1_ref, g_ref,
             bt_ref, o_ref):
        agg = p_ref[0] + p_ref[1]
        x = (
            jnp.dot(agg, we_ref[...], preferred_element_type=jnp.float32)
            + jnp.dot(v_ref[...], wv_ref[...], preferred_element_type=jnp.float32)
            + b0_ref[...]
        )
        x = jnp.maximum(x, 0.0)
        x = jnp.dot(x, w1_ref[...], preferred_element_type=jnp.float32) + b1_ref[...]
        x = jnp.maximum(x, 0.0)
        mu = jnp.mean(x, axis=-1, keepdims=True)
        xc = x - mu
        var = jnp.mean(xc * xc, axis=-1, keepdims=True)
        o_ref[...] = xc * jax.lax.rsqrt(var + 1e-5) * g_ref[...] + bt_ref[...]

    full = lambda i: (0, 0)
    return pl.pallas_call(
        body,
        grid=grid,
        in_specs=[
            pl.BlockSpec((NC, block_rows, h), lambda i: (0, i, 0)),
            pl.BlockSpec((block_rows, h), lambda i: (i, 0)),
            pl.BlockSpec((h, h), full),
            pl.BlockSpec((h, h), full),
            pl.BlockSpec((1, h), full),
            pl.BlockSpec((h, h), full),
            pl.BlockSpec((1, h), full),
            pl.BlockSpec((1, h), full),
            pl.BlockSpec((1, h), full),
        ],
        out_specs=pl.BlockSpec((block_rows, h), lambda i: (i, 0)),
        out_shape=jax.ShapeDtypeStruct((n, h), jnp.float32),
    )(partials, v, W_e, W_v, b0.reshape(1, h), W1, b1.reshape(1, h),
      gamma.reshape(1, h), beta.reshape(1, h))


@jax.jit
def kernel(v, edge_index, e, W_e, W_v, b0, W1, b1, gamma, beta):
    n, h = v.shape
    n_edges = e.shape[0]
    per_sub = n_edges // (NC * NS)
    ridx4 = edge_index[1].astype(jnp.int32).reshape(NC, NS, per_sub // CH, CH)
    n_pad = ((n + NS * 8 - 1) // (NS * 8)) * (NS * 8)
    zeros_nh = np.zeros((n_pad, h), np.float32)
    partials = _sc_scatter_add(e, ridx4, zeros_nh, n)
    return _tc_mlp(partials, v, W_e, W_v, b0, W1, b1, gamma, beta)


# speedup vs baseline: 7.7392x; 1.0178x over previous
import functools

import jax
import jax.numpy as jnp
import numpy as np
from jax import lax
from jax.experimental import pallas as pl
from jax.experimental.pallas import tpu as pltpu
from jax.experimental.pallas import tpu_sc as plsc

NC = 2
NS = 16
CH = 80


NBUF = 4


def _sc_scatter_add(e, ridx, zeros_nh, n_nodes):
    n_edges, h = e.shape
    per_sub = n_edges // (NC * NS)
    n_chunks = per_sub // CH
    n_pad = ((n_nodes + NS * 8 - 1) // (NS * 8)) * (NS * 8)
    rows_per_sub = n_pad // NS

    mesh = plsc.VectorSubcoreMesh(
        core_axis_name="c", subcore_axis_name="s", num_cores=NC, num_subcores=NS
    )

    @functools.partial(
        pl.kernel,
        out_type=jax.ShapeDtypeStruct((NC, n_pad, h), jnp.float32),
        mesh=mesh,
        scratch_types=[
            pltpu.VMEM_SHARED((n_pad, h), jnp.float32),
            pltpu.VMEM((NBUF, CH, h), jnp.float32),
            pltpu.VMEM((NBUF, CH), jnp.int32),
            pltpu.SemaphoreType.DMA,
            pltpu.SemaphoreType.DMA,
            pltpu.SemaphoreType.DMA,
        ],
    )
    def sc_kernel(e_hbm, ridx_hbm, zeros_hbm, out_hbm, acc_sh, bbuf, ibuf,
                  lsem, isem, ssem):
        c = lax.axis_index("c")
        s = lax.axis_index("s")
        base_e = (c * NS + s) * per_sub
        row0 = s * rows_per_sub

        def start_load(k, buf):
            pltpu.async_copy(
                e_hbm.at[pl.ds(base_e + k * CH, CH)], bbuf.at[buf], lsem
            )
            pltpu.async_copy(
                ridx_hbm.at[pl.ds(base_e + k * CH, CH)], ibuf.at[buf], isem
            )

        def wait_load(buf):
            pltpu.make_async_copy(
                e_hbm.at[pl.ds(base_e, CH)], bbuf.at[buf], lsem
            ).wait()
            pltpu.make_async_copy(
                ridx_hbm.at[pl.ds(base_e, CH)], ibuf.at[buf], isem
            ).wait()

        def wait_scatter_one():
            pltpu.make_async_copy(
                e_hbm.at[pl.ds(base_e, CH)], bbuf.at[0], ssem
            ).wait()

        start_load(0, 0)
        start_load(1, 1)
        pltpu.sync_copy(
            zeros_hbm.at[pl.ds(row0, rows_per_sub)],
            acc_sh.at[pl.ds(row0, rows_per_sub)],
        )
        plsc.subcore_barrier()

        def body(k, _):
            buf = lax.rem(k, NBUF)
            wait_load(buf)

            @pl.when(k >= 2)
            def _():
                wait_scatter_one()

            @pl.when(k + 2 < n_chunks)
            def _():
                start_load(k + 2, lax.rem(k + 2, NBUF))

            pltpu.async_copy(
                bbuf.at[buf], acc_sh.at[ibuf.at[buf]], ssem, add=True
            )
            return 0

        lax.fori_loop(0, n_chunks, body, 0)
        wait_scatter_one()
        wait_scatter_one()
        plsc.subcore_barrier()

        pltpu.sync_copy(
            acc_sh.at[pl.ds(row0, rows_per_sub)],
            out_hbm.at[c, pl.ds(row0, rows_per_sub)],
        )

    return sc_kernel(e, ridx, zeros_nh)


def _tc_mlp(partials, v, W_e, W_v, b0, W1, b1, gamma, beta, block_rows=1000):
    n, h = v.shape
    grid = (n // block_rows,)

    def body(p_ref, v_ref, we_ref, wv_ref, b0_ref, w1_ref, b1_ref, g_ref,
             bt_ref, o_ref):
        agg = p_ref[0] + p_ref[1]
        x = (
            jnp.dot(agg, we_ref[...], preferred_element_type=jnp.float32)
            + jnp.dot(v_ref[...], wv_ref[...], preferred_element_type=jnp.float32)
            + b0_ref[...]
        )
        x = jnp.maximum(x, 0.0)
        x = jnp.dot(x, w1_ref[...], preferred_element_type=jnp.float32) + b1_ref[...]
        x = jnp.maximum(x, 0.0)
        mu = jnp.mean(x, axis=-1, keepdims=True)
        xc = x - mu
        var = jnp.mean(xc * xc, axis=-1, keepdims=True)
        o_ref[...] = xc * jax.lax.rsqrt(var + 1e-5) * g_ref[...] + bt_ref[...]

    full = lambda i: (0, 0)
    return pl.pallas_call(
        body,
        grid=grid,
        in_specs=[
            pl.BlockSpec((NC, block_rows, h), lambda i: (0, i, 0)),
            pl.BlockSpec((block_rows, h), lambda i: (i, 0)),
            pl.BlockSpec((h, h), full),
            pl.BlockSpec((h, h), full),
            pl.BlockSpec((1, h), full),
            pl.BlockSpec((h, h), full),
            pl.BlockSpec((1, h), full),
            pl.BlockSpec((1, h), full),
            pl.BlockSpec((1, h), full),
        ],
        out_specs=pl.BlockSpec((block_rows, h), lambda i: (i, 0)),
        out_shape=jax.ShapeDtypeStruct((n, h), jnp.float32),
    )(partials, v, W_e, W_v, b0.reshape(1, h), W1, b1.reshape(1, h),
      gamma.reshape(1, h), beta.reshape(1, h))


@jax.jit
def kernel(v, edge_index, e, W_e, W_v, b0, W1, b1, gamma, beta):
    n, h = v.shape
    ridx = edge_index[1].astype(jnp.int32)
    n_pad = ((n + NS * 8 - 1) // (NS * 8)) * (NS * 8)
    zeros_nh = np.zeros((n_pad, h), np.float32)
    partials = _sc_scatter_add(e, ridx, zeros_nh, n)
    return _tc_mlp(partials, v, W_e, W_v, b0, W1, b1, gamma, beta)


# speedup vs baseline: 9.0276x; 1.1665x over previous
import functools

import jax
import jax.numpy as jnp
import numpy as np
from jax import lax
from jax.experimental import pallas as pl
from jax.experimental.pallas import tpu as pltpu
from jax.experimental.pallas import tpu_sc as plsc

NC = 2
NS = 16
NW = NC * NS
CH = 128
NBUF = 3


def _sc_scatter_add(e, edge_index, zeros_nh, n_nodes):
    n_edges, h = e.shape
    n_chunks = n_edges // CH
    n_base = n_chunks // NW
    n_extra = n_chunks - n_base * NW
    r_lo = (n_nodes // NS) // 8 * 8
    r_hi = n_nodes - r_lo * (NS - 1)

    mesh = plsc.VectorSubcoreMesh(
        core_axis_name="c", subcore_axis_name="s", num_cores=NC, num_subcores=NS
    )

    @functools.partial(
        pl.kernel,
        out_type=jax.ShapeDtypeStruct((NC, n_nodes, h), jnp.float32),
        mesh=mesh,
        scratch_types=[
            pltpu.VMEM_SHARED((n_nodes, h), jnp.float32),
            pltpu.VMEM((NBUF, CH, h), jnp.float32),
            pltpu.VMEM((NBUF, 2, CH), jnp.int32),
            pltpu.SemaphoreType.DMA,
            pltpu.SemaphoreType.DMA,
            pltpu.SemaphoreType.DMA,
        ],
    )
    def sc_kernel(e_hbm, ei_hbm, zeros_hbm, out_hbm, acc_sh, bbuf, ibuf,
                  lsem, isem, ssem):
        c = lax.axis_index("c")
        s = lax.axis_index("s")
        w = c * NS + s
        n_my = n_base + jnp.where(w < n_extra, 1, 0)

        def start_load(k, buf):
            cid = w + NW * k
            pltpu.async_copy(
                e_hbm.at[pl.ds(cid * CH, CH)], bbuf.at[buf], lsem
            )
            pltpu.async_copy(
                ei_hbm.at[pl.ds(0, 2), pl.ds(cid * CH, CH)], ibuf.at[buf], isem
            )

        def wait_load(buf):
            pltpu.make_async_copy(
                e_hbm.at[pl.ds(0, CH)], bbuf.at[buf], lsem
            ).wait()
            pltpu.make_async_copy(
                ei_hbm.at[pl.ds(0, 2), pl.ds(0, CH)], ibuf.at[buf], isem
            ).wait()

        def wait_scatter_one():
            pltpu.make_async_copy(
                e_hbm.at[pl.ds(0, CH)], bbuf.at[0], ssem
            ).wait()

        start_load(0, 0)
        start_load(1, 1)

        @pl.when(s < NS - 1)
        def _():
            pltpu.sync_copy(
                zeros_hbm.at[pl.ds(s * r_lo, r_lo)],
                acc_sh.at[pl.ds(s * r_lo, r_lo)],
            )

        @pl.when(s == NS - 1)
        def _():
            pltpu.sync_copy(
                zeros_hbm.at[pl.ds((NS - 1) * r_lo, r_hi)],
                acc_sh.at[pl.ds((NS - 1) * r_lo, r_hi)],
            )

        plsc.subcore_barrier()

        def body(k, _):
            buf = lax.rem(k, NBUF)
            wait_load(buf)

            @pl.when(k >= 1)
            def _():
                wait_scatter_one()

            @pl.when(k + 2 < n_my)
            def _():
                start_load(k + 2, lax.rem(k + 2, NBUF))

            pltpu.async_copy(
                bbuf.at[buf], acc_sh.at[ibuf.at[buf, 1]], ssem, add=True
            )
            return 0

        lax.fori_loop(0, n_my, body, 0)
        wait_scatter_one()
        plsc.subcore_barrier()

        @pl.when(s < NS - 1)
        def _():
            pltpu.sync_copy(
                acc_sh.at[pl.ds(s * r_lo, r_lo)],
                out_hbm.at[c, pl.ds(s * r_lo, r_lo)],
            )

        @pl.when(s == NS - 1)
        def _():
            pltpu.sync_copy(
                acc_sh.at[pl.ds((NS - 1) * r_lo, r_hi)],
                out_hbm.at[c, pl.ds((NS - 1) * r_lo, r_hi)],
            )

    return sc_kernel(e, edge_index, zeros_nh)


def _tc_mlp(partials, v, W_e, W_v, b0, W1, b1, gamma, beta, block_rows=1000):
    n, h = v.shape
    grid = (n // block_rows,)

    def body(p_ref, v_ref, we_ref, wv_ref, b0_ref, w1_ref, b1_ref, g_ref,
             bt_ref, o_ref):
        agg = p_ref[0] + p_ref[1]
        x = (
            jnp.dot(agg, we_ref[...], preferred_element_type=jnp.float32)
            + jnp.dot(v_ref[...], wv_ref[...], preferred_element_type=jnp.float32)
            + b0_ref[...]
        )
        x = jnp.maximum(x, 0.0)
        x = jnp.dot(x, w1_ref[...], preferred_element_type=jnp.float32) + b1_ref[...]
        x = jnp.maximum(x, 0.0)
        mu = jnp.mean(x, axis=-1, keepdims=True)
        xc = x - mu
        var = jnp.mean(xc * xc, axis=-1, keepdims=True)
        o_ref[...] = xc * jax.lax.rsqrt(var + 1e-5) * g_ref[...] + bt_ref[...]

    full = lambda i: (0, 0)
    return pl.pallas_call(
        body,
        grid=grid,
        in_specs=[
            pl.BlockSpec((NC, block_rows, h), lambda i: (0, i, 0)),
            pl.BlockSpec((block_rows, h), lambda i: (i, 0)),
            pl.BlockSpec((h, h), full),
            pl.BlockSpec((h, h), full),
            pl.BlockSpec((1, h), full),
            pl.BlockSpec((h, h), full),
            pl.BlockSpec((1, h), full),
            pl.BlockSpec((1, h), full),
            pl.BlockSpec((1, h), full),
        ],
        out_specs=pl.BlockSpec((block_rows, h), lambda i: (i, 0)),
        out_shape=jax.ShapeDtypeStruct((n, h), jnp.float32),
    )(partials, v, W_e, W_v, b0.reshape(1, h), W1, b1.reshape(1, h),
      gamma.reshape(1, h), beta.reshape(1, h))


@jax.jit
def kernel(v, edge_index, e, W_e, W_v, b0, W1, b1, gamma, beta):
    n, h = v.shape
    ei = edge_index.astype(jnp.int32)
    zeros_nh = np.zeros((n, h), np.float32)
    partials = _sc_scatter_add(e, ei, zeros_nh, n)
    return _tc_mlp(partials, v, W_e, W_v, b0, W1, b1, gamma, beta)


# speedup vs baseline: 9.2194x; 1.0212x over previous
import functools

import jax
import jax.numpy as jnp
import numpy as np
from jax import lax
from jax.experimental import pallas as pl
from jax.experimental.pallas import tpu as pltpu
from jax.experimental.pallas import tpu_sc as plsc

NC = 2
NS = 16
NW = NC * NS
CH = 128
NBUF = 3


def _sc_scatter_add(e, edge_index, zeros_nh, n_nodes):
    n_edges, h = e.shape
    n_chunks = n_edges // CH
    n_base = n_chunks // NW
    n_extra = n_chunks - n_base * NW
    r_lo = (n_nodes // NS) // 8 * 8
    r_hi = n_nodes - r_lo * (NS - 1)

    mesh = plsc.VectorSubcoreMesh(
        core_axis_name="c", subcore_axis_name="s", num_cores=NC, num_subcores=NS
    )

    @functools.partial(
        pl.kernel,
        out_type=jax.ShapeDtypeStruct((NC, n_nodes, h), jnp.float32),
        mesh=mesh,
        scratch_types=[
            pltpu.VMEM_SHARED((n_nodes, h), jnp.float32),
            pltpu.VMEM((NBUF, CH, h), jnp.float32),
            pltpu.VMEM((NBUF, 2, CH), jnp.int32),
            pltpu.SemaphoreType.DMA,
            pltpu.SemaphoreType.DMA,
            pltpu.SemaphoreType.DMA,
        ],
    )
    def sc_kernel(e_hbm, ei_hbm, zeros_hbm, out_hbm, acc_sh, bbuf, ibuf,
                  lsem, isem, ssem):
        c = lax.axis_index("c")
        s = lax.axis_index("s")
        w = c * NS + s
        n_my = n_base + jnp.where(w < n_extra, 1, 0)

        def start_load(k, buf):
            cid = w + NW * k
            pltpu.async_copy(
                e_hbm.at[pl.ds(cid * CH, CH)], bbuf.at[buf], lsem
            )
            pltpu.async_copy(
                ei_hbm.at[pl.ds(0, 2), pl.ds(cid * CH, CH)], ibuf.at[buf], isem
            )

        def wait_load(buf):
            pltpu.make_async_copy(
                e_hbm.at[pl.ds(0, CH)], bbuf.at[buf], lsem
            ).wait()
            pltpu.make_async_copy(
                ei_hbm.at[pl.ds(0, 2), pl.ds(0, CH)], ibuf.at[buf], isem
            ).wait()

        def wait_scatter_one():
            pltpu.make_async_copy(
                e_hbm.at[pl.ds(0, CH)], bbuf.at[0], ssem
            ).wait()

        start_load(0, 0)
        start_load(1, 1)

        @pl.when(s < NS - 1)
        def _():
            pltpu.sync_copy(
                zeros_hbm.at[pl.ds(s * r_lo, r_lo)],
                acc_sh.at[pl.ds(s * r_lo, r_lo)],
            )

        @pl.when(s == NS - 1)
        def _():
            pltpu.sync_copy(
                zeros_hbm.at[pl.ds((NS - 1) * r_lo, r_hi)],
                acc_sh.at[pl.ds((NS - 1) * r_lo, r_hi)],
            )

        plsc.subcore_barrier()

        def body(k, _):
            buf = lax.rem(k, NBUF)
            wait_load(buf)

            @pl.when(k >= 1)
            def _():
                wait_scatter_one()

            @pl.when(k + 2 < n_my)
            def _():
                start_load(k + 2, lax.rem(k + 2, NBUF))

            pltpu.async_copy(
                bbuf.at[buf], acc_sh.at[ibuf.at[buf, 1]], ssem, add=True
            )
            return 0

        lax.fori_loop(0, n_my, body, 0)
        wait_scatter_one()
        plsc.subcore_barrier()

        @pl.when(s < NS - 1)
        def _():
            pltpu.sync_copy(
                acc_sh.at[pl.ds(s * r_lo, r_lo)],
                out_hbm.at[c, pl.ds(s * r_lo, r_lo)],
            )

        @pl.when(s == NS - 1)
        def _():
            pltpu.sync_copy(
                acc_sh.at[pl.ds((NS - 1) * r_lo, r_hi)],
                out_hbm.at[c, pl.ds((NS - 1) * r_lo, r_hi)],
            )

    return sc_kernel(e, edge_index, zeros_nh)


def _tc_pre(v, W_v, b0, block_rows=2000):
    n, h = v.shape

    def body(v_ref, wv_ref, b0_ref, o_ref):
        o_ref[...] = (
            jnp.dot(v_ref[...], wv_ref[...], preferred_element_type=jnp.float32)
            + b0_ref[...]
        )

    full = lambda i: (0, 0)
    return pl.pallas_call(
        body,
        grid=(n // block_rows,),
        in_specs=[
            pl.BlockSpec((block_rows, h), lambda i: (i, 0)),
            pl.BlockSpec((h, h), full),
            pl.BlockSpec((1, h), full),
        ],
        out_specs=pl.BlockSpec((block_rows, h), lambda i: (i, 0)),
        out_shape=jax.ShapeDtypeStruct((n, h), jnp.float32),
    )(v, W_v, b0.reshape(1, h))


def _tc_mlp(partials, t, W_e, W1, b1, gamma, beta, block_rows=2000):
    n, h = t.shape
    grid = (n // block_rows,)

    def body(p_ref, t_ref, we_ref, w1_ref, b1_ref, g_ref, bt_ref, o_ref):
        agg = p_ref[0] + p_ref[1]
        x = (
            jnp.dot(agg, we_ref[...], preferred_element_type=jnp.float32)
            + t_ref[...]
        )
        x = jnp.maximum(x, 0.0)
        x = jnp.dot(x, w1_ref[...], preferred_element_type=jnp.float32) + b1_ref[...]
        x = jnp.maximum(x, 0.0)
        mu = jnp.mean(x, axis=-1, keepdims=True)
        xc = x - mu
        var = jnp.mean(xc * xc, axis=-1, keepdims=True)
        o_ref[...] = xc * jax.lax.rsqrt(var + 1e-5) * g_ref[...] + bt_ref[...]

    full = lambda i: (0, 0)
    return pl.pallas_call(
        body,
        grid=grid,
        in_specs=[
            pl.BlockSpec((NC, block_rows, h), lambda i: (0, i, 0)),
            pl.BlockSpec((block_rows, h), lambda i: (i, 0)),
            pl.BlockSpec((h, h), full),
            pl.BlockSpec((h, h), full),
            pl.BlockSpec((1, h), full),
            pl.BlockSpec((1, h), full),
            pl.BlockSpec((1, h), full),
        ],
        out_specs=pl.BlockSpec((block_rows, h), lambda i: (i, 0)),
        out_shape=jax.ShapeDtypeStruct((n, h), jnp.float32),
    )(partials, t, W_e, W1, b1.reshape(1, h),
      gamma.reshape(1, h), beta.reshape(1, h))


@jax.jit
def kernel(v, edge_index, e, W_e, W_v, b0, W1, b1, gamma, beta):
    n, h = v.shape
    ei = edge_index.astype(jnp.int32)
    zeros_nh = np.zeros((n, h), np.float32)
    partials = _sc_scatter_add(e, ei, zeros_nh, n)
    t = _tc_pre(v, W_v, b0)
    return _tc_mlp(partials, t, W_e, W1, b1, gamma, beta)


# speedup vs baseline: 9.2240x; 1.0005x over previous
import functools

import jax
import jax.numpy as jnp
import numpy as np
from jax import lax
from jax.experimental import pallas as pl
from jax.experimental.pallas import tpu as pltpu
from jax.experimental.pallas import tpu_sc as plsc

NC = 2
NS = 16
NW = NC * NS
CH = 128
NBUF = 3


def _sc_scatter_add(e, edge_index, zeros_nh, n_nodes):
    n_edges, h = e.shape
    n_chunks = n_edges // CH
    n_base = n_chunks // NW
    n_extra = n_chunks - n_base * NW
    r_lo = (n_nodes // NS) // 8 * 8
    r_hi = n_nodes - r_lo * (NS - 1)

    mesh = plsc.VectorSubcoreMesh(
        core_axis_name="c", subcore_axis_name="s", num_cores=NC, num_subcores=NS
    )

    @functools.partial(
        pl.kernel,
        out_type=jax.ShapeDtypeStruct((NC, n_nodes, h), jnp.float32),
        mesh=mesh,
        scratch_types=[
            pltpu.VMEM_SHARED((n_nodes, h), jnp.float32),
            pltpu.VMEM((NBUF, CH, h), jnp.float32),
            pltpu.VMEM((NBUF, 2, CH), jnp.int32),
            pltpu.SemaphoreType.DMA,
            pltpu.SemaphoreType.DMA,
            pltpu.SemaphoreType.DMA,
        ],
    )
    def sc_kernel(e_hbm, ei_hbm, zeros_hbm, out_hbm, acc_sh, bbuf, ibuf,
                  lsem, isem, ssem):
        c = lax.axis_index("c")
        s = lax.axis_index("s")
        w = c * NS + s
        n_my = n_base + jnp.where(w < n_extra, 1, 0)

        def start_load(k, buf):
            cid = w + NW * k
            pltpu.async_copy(
                e_hbm.at[pl.ds(cid * CH, CH)], bbuf.at[buf], lsem
            )
            pltpu.async_copy(
                ei_hbm.at[pl.ds(0, 2), pl.ds(cid * CH, CH)], ibuf.at[buf], isem
            )

        def wait_load(buf):
            pltpu.make_async_copy(
                e_hbm.at[pl.ds(0, CH)], bbuf.at[buf], lsem
            ).wait()
            pltpu.make_async_copy(
                ei_hbm.at[pl.ds(0, 2), pl.ds(0, CH)], ibuf.at[buf], isem
            ).wait()

        def wait_scatter_one():
            pltpu.make_async_copy(
                e_hbm.at[pl.ds(0, CH)], bbuf.at[0], ssem
            ).wait()

        start_load(0, 0)
        start_load(1, 1)

        @pl.when(s < NS - 1)
        def _():
            pltpu.sync_copy(
                zeros_hbm.at[pl.ds(s * r_lo, r_lo)],
                acc_sh.at[pl.ds(s * r_lo, r_lo)],
            )

        @pl.when(s == NS - 1)
        def _():
            pltpu.sync_copy(
                zeros_hbm.at[pl.ds((NS - 1) * r_lo, r_hi)],
                acc_sh.at[pl.ds((NS - 1) * r_lo, r_hi)],
            )

        plsc.subcore_barrier()

        def body(k, _):
            buf = lax.rem(k, NBUF)
            wait_load(buf)

            @pl.when(k >= 1)
            def _():
                wait_scatter_one()

            @pl.when(k + 2 < n_my)
            def _():
                start_load(k + 2, lax.rem(k + 2, NBUF))

            pltpu.async_copy(
                bbuf.at[buf], acc_sh.at[ibuf.at[buf, 1]], ssem, add=True
            )
            return 0

        lax.fori_loop(0, n_my, body, 0)
        wait_scatter_one()
        plsc.subcore_barrier()

        @pl.when(s < NS - 1)
        def _():
            pltpu.sync_copy(
                acc_sh.at[pl.ds(s * r_lo, r_lo)],
                out_hbm.at[c, pl.ds(s * r_lo, r_lo)],
            )

        @pl.when(s == NS - 1)
        def _():
            pltpu.sync_copy(
                acc_sh.at[pl.ds((NS - 1) * r_lo, r_hi)],
                out_hbm.at[c, pl.ds((NS - 1) * r_lo, r_hi)],
            )

    return sc_kernel(e, edge_index, zeros_nh)


def _tc_pre(v, W_v, b0, block_rows=2000):
    n, h = v.shape

    def body(v_ref, wv_ref, b0_ref, o_ref):
        o_ref[...] = (
            jnp.dot(v_ref[...], wv_ref[...], preferred_element_type=jnp.float32)
            + b0_ref[...]
        )

    full = lambda i: (0, 0)
    return pl.pallas_call(
        body,
        grid=(n // block_rows,),
        in_specs=[
            pl.BlockSpec((block_rows, h), lambda i: (i, 0)),
            pl.BlockSpec((h, h), full),
            pl.BlockSpec((1, h), full),
        ],
        out_specs=pl.BlockSpec((block_rows, h), lambda i: (i, 0)),
        out_shape=jax.ShapeDtypeStruct((n, h), jnp.float32),
    )(v, W_v, b0.reshape(1, h))


def _tc_mlp(partials, t, W_e, W1, b1, gamma, beta, block_rows=2000):
    n, h = t.shape
    grid = (n // block_rows,)

    def body(p_ref, t_ref, we_ref, w1_ref, b1_ref, g_ref, bt_ref, o_ref):
        agg = p_ref[0] + p_ref[1]
        x = (
            jnp.dot(agg, we_ref[...], preferred_element_type=jnp.float32)
            + t_ref[...]
        )
        x = jnp.maximum(x, 0.0)
        x = jnp.dot(x, w1_ref[...], preferred_element_type=jnp.float32) + b1_ref[...]
        x = jnp.maximum(x, 0.0)
        mu = jnp.mean(x, axis=-1, keepdims=True)
        xc = x - mu
        var = jnp.mean(xc * xc, axis=-1, keepdims=True)
        o_ref[...] = xc * jax.lax.rsqrt(var + 1e-5) * g_ref[...] + bt_ref[...]

    full = lambda i: (0, 0)
    return pl.pallas_call(
        body,
        grid=grid,
        in_specs=[
            pl.BlockSpec((NC, block_rows, h), lambda i: (0, i, 0)),
            pl.BlockSpec((block_rows, h), lambda i: (i, 0)),
            pl.BlockSpec((h, h), full),
            pl.BlockSpec((h, h), full),
            pl.BlockSpec((1, h), full),
            pl.BlockSpec((1, h), full),
            pl.BlockSpec((1, h), full),
        ],
        out_specs=pl.BlockSpec((block_rows, h), lambda i: (i, 0)),
        out_shape=jax.ShapeDtypeStruct((n, h), jnp.float32),
    )(partials, t, W_e, W1, b1.reshape(1, h),
      gamma.reshape(1, h), beta.reshape(1, h))


@jax.jit
def kernel(v, edge_index, e, W_e, W_v, b0, W1, b1, gamma, beta):
    n, h = v.shape
    ei = edge_index.astype(jnp.int32)
    zeros_nh = jnp.zeros((n, h), jnp.float32)
    partials = _sc_scatter_add(e, ei, zeros_nh, n)
    t = _tc_pre(v, W_v, b0)
    return _tc_mlp(partials, t, W_e, W1, b1, gamma, beta)
